# Initial kernel scaffold; baseline (speedup 1.0000x reference)
import jax
import jax.numpy as jnp
from jax.experimental import pallas as pl


def kernel(users, items, W_user, W_item, W_user_cross, W_item_cross):
    raise NotImplementedError("write your pallas kernel here")


# speedup vs baseline: 1.1698x; 1.1698x over previous
import functools

import jax
import jax.numpy as jnp
from jax import lax
from jax.experimental import pallas as pl
from jax.experimental.pallas import tpu as pltpu
from jax.experimental.pallas import tpu_sc as plsc

NC = 2
NS = 16
L = 16
NW = NC * NS
B = 16384
D = 128
BPW = B // NW
CH = 256
NCHUNK = BPW // CH


def _sc_body(users, items, w_u, w_i, w_uc, w_ic,
             out_u, out_i, out_cu, out_ci, out_x,
             idx_u, idx_i, buf_a, buf_b, xbuf, sem):
    wid = lax.axis_index("s") * NC + lax.axis_index("c")
    base = wid * BPW
    for c in range(NCHUNK):
        off = base + c * CH
        pltpu.sync_copy(users.at[pl.ds(off, CH)], idx_u.at[c])
        pltpu.sync_copy(items.at[pl.ds(off, CH)], idx_i.at[c])
    for c in range(NCHUNK):
        off = base + c * CH
        pltpu.async_copy(w_u.at[idx_u.at[c]], buf_a, sem).wait()
        pltpu.sync_copy(buf_a, out_u.at[pl.ds(off, CH)])
        pltpu.async_copy(w_i.at[idx_i.at[c]], buf_a, sem).wait()
        pltpu.sync_copy(buf_a, out_i.at[pl.ds(off, CH)])
        pltpu.async_copy(w_uc.at[idx_u.at[c]], buf_a, sem).wait()
        pltpu.sync_copy(buf_a, out_cu.at[pl.ds(off, CH)])
        pltpu.async_copy(w_ic.at[idx_i.at[c]], buf_b, sem).wait()
        pltpu.sync_copy(buf_b, out_ci.at[pl.ds(off, CH)])

        def grp(g, carry):
            lanes = lax.iota(jnp.int32, L)
            vec = jnp.zeros((L,), jnp.float32)
            for k in range(L):
                r = g * L + k
                acc = buf_a[r, pl.ds(0, L)] * buf_b[r, pl.ds(0, L)]
                for j in range(1, D // L):
                    acc = acc + (buf_a[r, pl.ds(j * L, L)]
                                 * buf_b[r, pl.ds(j * L, L)])
                vec = jnp.where(lanes == k, jnp.sum(acc), vec)
            xbuf[pl.ds(c * CH + g * L, L)] = vec
            return carry

        lax.fori_loop(0, CH // L, grp, 0)
    pltpu.sync_copy(xbuf, out_x.at[pl.ds(base, BPW)])


_mesh = plsc.VectorSubcoreMesh(core_axis_name="c", subcore_axis_name="s")

_sc_call = functools.partial(
    pl.kernel,
    out_type=(
        jax.ShapeDtypeStruct((B, D), jnp.float32),
        jax.ShapeDtypeStruct((B, D), jnp.float32),
        jax.ShapeDtypeStruct((B, D), jnp.float32),
        jax.ShapeDtypeStruct((B, D), jnp.float32),
        jax.ShapeDtypeStruct((B,), jnp.float32),
    ),
    mesh=_mesh,
    compiler_params=pltpu.CompilerParams(
        needs_layout_passes=False, use_tc_tiling_on_sc=False),
    scratch_types=[
        pltpu.VMEM((NCHUNK, CH), jnp.int32),
        pltpu.VMEM((NCHUNK, CH), jnp.int32),
        pltpu.VMEM((CH, D), jnp.float32),
        pltpu.VMEM((CH, D), jnp.float32),
        pltpu.VMEM((BPW,), jnp.float32),
        pltpu.SemaphoreType.DMA,
    ],
)(_sc_body)


@jax.jit
def kernel(users, items, W_user, W_item, W_user_cross, W_item_cross):
    out_u, out_i, out_cu, out_ci, out_x = _sc_call(
        users, items, W_user, W_item, W_user_cross, W_item_cross)
    return out_u, out_i, out_cu, out_ci, out_x.reshape(B, 1)


# speedup vs baseline: 1.3071x; 1.1174x over previous
import functools

import jax
import jax.numpy as jnp
from jax import lax
from jax.experimental import pallas as pl
from jax.experimental.pallas import tpu as pltpu
from jax.experimental.pallas import tpu_sc as plsc

NC = 2
NS = 16
L = 16
NW = NC * NS
B = 16384
D = 128
BPW = B // NW
CH = 128
NCHUNK = BPW // CH


def _sc_body(users, items, w_u, w_i, w_uc, w_ic,
             out_u, out_i, out_cu, out_ci, out_x,
             idx_u, idx_i, buf_u, buf_i, buf_uc, buf_ic, xbuf,
             sg_u, sg_i, sg_uc, sg_ic, sw_u, sw_i, sw_uc, sw_ic):
    wid = lax.axis_index("s") * NC + lax.axis_index("c")
    base = wid * BPW
    for c in range(NCHUNK):
        off = base + c * CH
        pltpu.sync_copy(users.at[pl.ds(off, CH)], idx_u.at[c])
        pltpu.sync_copy(items.at[pl.ds(off, CH)], idx_i.at[c])

    wbs = []
    for c in range(NCHUNK):
        off = base + c * CH
        for wb in wbs:
            wb.wait()
        g_u = pltpu.async_copy(w_u.at[idx_u.at[c]], buf_u, sg_u)
        g_i = pltpu.async_copy(w_i.at[idx_i.at[c]], buf_i, sg_i)
        g_uc = pltpu.async_copy(w_uc.at[idx_u.at[c]], buf_uc, sg_uc)
        g_ic = pltpu.async_copy(w_ic.at[idx_i.at[c]], buf_ic, sg_ic)
        g_u.wait()
        wb_u = pltpu.async_copy(buf_u, out_u.at[pl.ds(off, CH)], sw_u)
        g_i.wait()
        wb_i = pltpu.async_copy(buf_i, out_i.at[pl.ds(off, CH)], sw_i)
        g_uc.wait()
        wb_uc = pltpu.async_copy(buf_uc, out_cu.at[pl.ds(off, CH)], sw_uc)
        g_ic.wait()
        wb_ic = pltpu.async_copy(buf_ic, out_ci.at[pl.ds(off, CH)], sw_ic)
        wbs = [wb_u, wb_i, wb_uc, wb_ic]

        def grp(g, carry):
            lanes = lax.iota(jnp.int32, L)
            vec = jnp.zeros((L,), jnp.float32)
            for k in range(L):
                r = g * L + k
                acc = buf_uc[r, pl.ds(0, L)] * buf_ic[r, pl.ds(0, L)]
                for j in range(1, D // L):
                    acc = acc + (buf_uc[r, pl.ds(j * L, L)]
                                 * buf_ic[r, pl.ds(j * L, L)])
                vec = jnp.where(lanes == k, jnp.sum(acc), vec)
            xbuf[pl.ds(c * CH + g * L, L)] = vec
            return carry

        lax.fori_loop(0, CH // L, grp, 0)

    for wb in wbs:
        wb.wait()
    pltpu.sync_copy(xbuf, out_x.at[pl.ds(base, BPW)])


_mesh = plsc.VectorSubcoreMesh(core_axis_name="c", subcore_axis_name="s")

_sc_call = functools.partial(
    pl.kernel,
    out_type=(
        jax.ShapeDtypeStruct((B, D), jnp.float32),
        jax.ShapeDtypeStruct((B, D), jnp.float32),
        jax.ShapeDtypeStruct((B, D), jnp.float32),
        jax.ShapeDtypeStruct((B, D), jnp.float32),
        jax.ShapeDtypeStruct((B,), jnp.float32),
    ),
    mesh=_mesh,
    compiler_params=pltpu.CompilerParams(
        needs_layout_passes=False, use_tc_tiling_on_sc=False),
    scratch_types=[
        pltpu.VMEM((NCHUNK, CH), jnp.int32),
        pltpu.VMEM((NCHUNK, CH), jnp.int32),
        pltpu.VMEM((CH, D), jnp.float32),
        pltpu.VMEM((CH, D), jnp.float32),
        pltpu.VMEM((CH, D), jnp.float32),
        pltpu.VMEM((CH, D), jnp.float32),
        pltpu.VMEM((BPW,), jnp.float32),
        pltpu.SemaphoreType.DMA,
        pltpu.SemaphoreType.DMA,
        pltpu.SemaphoreType.DMA,
        pltpu.SemaphoreType.DMA,
        pltpu.SemaphoreType.DMA,
        pltpu.SemaphoreType.DMA,
        pltpu.SemaphoreType.DMA,
        pltpu.SemaphoreType.DMA,
    ],
)(_sc_body)


@jax.jit
def kernel(users, items, W_user, W_item, W_user_cross, W_item_cross):
    out_u, out_i, out_cu, out_ci, out_x = _sc_call(
        users, items, W_user, W_item, W_user_cross, W_item_cross)
    return out_u, out_i, out_cu, out_ci, out_x.reshape(B, 1)


# speedup vs baseline: 1.4552x; 1.1133x over previous
import functools

import jax
import jax.numpy as jnp
from jax import lax
from jax.experimental import pallas as pl
from jax.experimental.pallas import tpu as pltpu
from jax.experimental.pallas import tpu_sc as plsc

NC = 2
NS = 16
L = 16
NW = NC * NS
B = 16384
D = 128
BPW = B // NW
CH = 64
NCHUNK = BPW // CH


def _sc_body(users, items, w_u, w_i, w_uc, w_ic,
             out_u, out_i, out_cu, out_ci, out_x,
             idx_u, idx_i,
             buf_u0, buf_i0, buf_uc0, buf_ic0,
             buf_u1, buf_i1, buf_uc1, buf_ic1,
             xbuf, *sems):
    wid = lax.axis_index("s") * NC + lax.axis_index("c")
    base = wid * BPW
    pltpu.sync_copy(users.at[pl.ds(base, BPW)], idx_u)
    pltpu.sync_copy(items.at[pl.ds(base, BPW)], idx_i)

    bufs = [(buf_u0, buf_i0, buf_uc0, buf_ic0),
            (buf_u1, buf_i1, buf_uc1, buf_ic1)]
    sg = [sems[0:4], sems[4:8]]
    sw = [sems[8:12], sems[12:16]]
    outs = (out_u, out_i, out_cu, out_ci)

    def fire_gathers(c, s):
        iu = idx_u.at[pl.ds(c * CH, CH)]
        ii = idx_i.at[pl.ds(c * CH, CH)]
        bu, bi, buc, bic = bufs[s]
        return [
            pltpu.async_copy(w_u.at[iu], bu, sg[s][0]),
            pltpu.async_copy(w_i.at[ii], bi, sg[s][1]),
            pltpu.async_copy(w_uc.at[iu], buc, sg[s][2]),
            pltpu.async_copy(w_ic.at[ii], bic, sg[s][3]),
        ]

    def drain_and_compute(c, s, gh):
        off = base + c * CH
        wb = []
        for t in range(4):
            gh[t].wait()
            wb.append(pltpu.async_copy(
                bufs[s][t], outs[t].at[pl.ds(off, CH)], sw[s][t]))
        buc, bic = bufs[s][2], bufs[s][3]

        def grp(g, carry):
            lanes = lax.iota(jnp.int32, L)
            vec = jnp.zeros((L,), jnp.float32)
            for k in range(L):
                r = g * L + k
                acc = buc[r, pl.ds(0, L)] * bic[r, pl.ds(0, L)]
                for j in range(1, D // L):
                    acc = acc + (buc[r, pl.ds(j * L, L)]
                                 * bic[r, pl.ds(j * L, L)])
                vec = jnp.where(lanes == k, jnp.sum(acc), vec)
            xbuf[pl.ds(c * CH + g * L, L)] = vec
            return carry

        lax.fori_loop(0, CH // L, grp, 0)
        return wb

    gh = [None, None]
    wbh = [[], []]
    for c in range(NCHUNK + 1):
        s = c % 2
        if c < NCHUNK:
            for h in wbh[s]:
                h.wait()
            wbh[s] = []
            gh[s] = fire_gathers(c, s)
        if c >= 1:
            p = (c - 1) % 2
            wbh[p] = drain_and_compute(c - 1, p, gh[p])

    for s in range(2):
        for h in wbh[s]:
            h.wait()
    pltpu.sync_copy(xbuf, out_x.at[pl.ds(base, BPW)])


_mesh = plsc.VectorSubcoreMesh(core_axis_name="c", subcore_axis_name="s")

_sc_call = functools.partial(
    pl.kernel,
    out_type=(
        jax.ShapeDtypeStruct((B, D), jnp.float32),
        jax.ShapeDtypeStruct((B, D), jnp.float32),
        jax.ShapeDtypeStruct((B, D), jnp.float32),
        jax.ShapeDtypeStruct((B, D), jnp.float32),
        jax.ShapeDtypeStruct((B,), jnp.float32),
    ),
    mesh=_mesh,
    compiler_params=pltpu.CompilerParams(
        needs_layout_passes=False, use_tc_tiling_on_sc=False),
    scratch_types=(
        [pltpu.VMEM((BPW,), jnp.int32)] * 2
        + [pltpu.VMEM((CH, D), jnp.float32)] * 8
        + [pltpu.VMEM((BPW,), jnp.float32)]
        + [pltpu.SemaphoreType.DMA] * 16
    ),
)(_sc_body)


@jax.jit
def kernel(users, items, W_user, W_item, W_user_cross, W_item_cross):
    out_u, out_i, out_cu, out_ci, out_x = _sc_call(
        users, items, W_user, W_item, W_user_cross, W_item_cross)
    return out_u, out_i, out_cu, out_ci, out_x.reshape(B, 1)


# speedup vs baseline: 1.5411x; 1.0590x over previous
import functools

import jax
import jax.numpy as jnp
from jax import lax
from jax.experimental import pallas as pl
from jax.experimental.pallas import tpu as pltpu
from jax.experimental.pallas import tpu_sc as plsc

NC = 2
NS = 16
L = 16
NW = NC * NS
B = 16384
D = 128
BPW = B // NW
CH = 64
NCHUNK = BPW // CH


def _sc_body(users, items, w_u, w_i, w_uc, w_ic,
             out_u, out_i, out_cu, out_ci, out_x,
             idx_u, idx_i,
             buf_u0, buf_i0, buf_uc0, buf_ic0,
             buf_u1, buf_i1, buf_uc1, buf_ic1,
             xbuf, *sems):
    wid = lax.axis_index("s") * NC + lax.axis_index("c")
    base = wid * BPW
    pltpu.sync_copy(users.at[pl.ds(base, BPW)], idx_u)
    pltpu.sync_copy(items.at[pl.ds(base, BPW)], idx_i)

    bufs = [(buf_u0, buf_i0, buf_uc0, buf_ic0),
            (buf_u1, buf_i1, buf_uc1, buf_ic1)]
    sg = [sems[0:4], sems[4:8]]
    sw = [sems[8:12], sems[12:16]]
    outs = (out_u, out_i, out_cu, out_ci)

    def fire_gathers(c, s):
        iu = idx_u.at[pl.ds(c * CH, CH)]
        ii = idx_i.at[pl.ds(c * CH, CH)]
        bu, bi, buc, bic = bufs[s]
        pltpu.async_copy(w_u.at[iu], bu, sg[s][0])
        pltpu.async_copy(w_i.at[ii], bi, sg[s][1])
        pltpu.async_copy(w_uc.at[iu], buc, sg[s][2])
        pltpu.async_copy(w_ic.at[ii], bic, sg[s][3])

    def wait_gather(s, t):
        pltpu.make_async_copy(w_u.at[idx_u.at[pl.ds(0, CH)]],
                              bufs[s][t], sg[s][t]).wait()

    def fire_wb(c, s, t):
        off = base + c * CH
        pltpu.async_copy(bufs[s][t], outs[t].at[pl.ds(off, CH)], sw[s][t])

    def wait_wb(s, t):
        pltpu.make_async_copy(bufs[s][t],
                              outs[t].at[pl.ds(base, CH)], sw[s][t]).wait()

    def compute_cross(c, s):
        buc, bic = bufs[s][2], bufs[s][3]

        def grp16(g, carry):
            lanes = lax.iota(jnp.int32, L)
            vec = jnp.zeros((L,), jnp.float32)
            for k in range(L):
                r = g * L + k
                acc = buc[r, pl.ds(0, L)] * bic[r, pl.ds(0, L)]
                for j in range(1, D // L):
                    acc = acc + (buc[r, pl.ds(j * L, L)]
                                 * bic[r, pl.ds(j * L, L)])
                vec = jnp.where(lanes == k, jnp.sum(acc), vec)
            xbuf[pl.ds(c * CH + g * L, L)] = vec
            return carry

        lax.fori_loop(0, CH // L, grp16, 0)

    fire_gathers(0, 0)
    fire_gathers(1, 1)
    for t in (2, 3):
        wait_gather(0, t)
        fire_wb(0, 0, t)
    compute_cross(0, 0)
    for t in (0, 1):
        wait_gather(0, t)
        fire_wb(0, 0, t)

    def pair(it, carry):
        c0 = 2 * it
        c1 = c0 + 1
        for t in range(4):
            wait_wb(0, t)
        fire_gathers(c0, 0)
        for t in (2, 3):
            wait_gather(1, t)
            fire_wb(c0 - 1, 1, t)
        compute_cross(c0 - 1, 1)
        for t in (0, 1):
            wait_gather(1, t)
            fire_wb(c0 - 1, 1, t)
        for t in range(4):
            wait_wb(1, t)
        fire_gathers(c1, 1)
        for t in (2, 3):
            wait_gather(0, t)
            fire_wb(c0, 0, t)
        compute_cross(c0, 0)
        for t in (0, 1):
            wait_gather(0, t)
            fire_wb(c0, 0, t)
        return carry

    lax.fori_loop(1, NCHUNK // 2, pair, 0)

    for t in (2, 3):
        wait_gather(1, t)
        fire_wb(NCHUNK - 1, 1, t)
    compute_cross(NCHUNK - 1, 1)
    for t in (0, 1):
        wait_gather(1, t)
        fire_wb(NCHUNK - 1, 1, t)
    for s in range(2):
        for t in range(4):
            wait_wb(s, t)
    pltpu.sync_copy(xbuf, out_x.at[pl.ds(base, BPW)])


_mesh = plsc.VectorSubcoreMesh(core_axis_name="c", subcore_axis_name="s")

_sc_call = functools.partial(
    pl.kernel,
    out_type=(
        jax.ShapeDtypeStruct((B, D), jnp.float32),
        jax.ShapeDtypeStruct((B, D), jnp.float32),
        jax.ShapeDtypeStruct((B, D), jnp.float32),
        jax.ShapeDtypeStruct((B, D), jnp.float32),
        jax.ShapeDtypeStruct((B,), jnp.float32),
    ),
    mesh=_mesh,
    compiler_params=pltpu.CompilerParams(
        needs_layout_passes=False, use_tc_tiling_on_sc=False),
    scratch_types=(
        [pltpu.VMEM((BPW,), jnp.int32)] * 2
        + [pltpu.VMEM((CH, D), jnp.float32)] * 8
        + [pltpu.VMEM((BPW,), jnp.float32)]
        + [pltpu.SemaphoreType.DMA] * 16
    ),
)(_sc_body)


@jax.jit
def kernel(users, items, W_user, W_item, W_user_cross, W_item_cross):
    out_u, out_i, out_cu, out_ci, out_x = _sc_call(
        users, items, W_user, W_item, W_user_cross, W_item_cross)
    return out_u, out_i, out_cu, out_ci, out_x.reshape(B, 1)
